# baseline (speedup 1.0000x reference)
import jax
import jax.numpy as jnp
from jax import lax
from jax.experimental import pallas as pl
from jax.experimental.pallas import tpu as pltpu
from jax.experimental.pallas import tpu_sc as plsc

N = 100000
D = 128
B = 256
HIDDEN = 256

CHUNK = 128
NUM_FULL = N // CHUNK
TAIL = N - NUM_FULL * CHUNK
NC = 2
NS = 16
NW = NC * NS
SLAB = 25
SLAB_ROWS = SLAB * CHUNK
LAST_CHUNKS = NUM_FULL - (NW - 1) * SLAB
ROWS_PER_TILE = B // NS


def _seg_sum_body(x_hbm, e_hbm, b2d_hbm, b1d_hbm, ones_hbm,
                  node_out, edge_out, cnt_out,
                  xbuf0, xbuf1, ebuf0, ebuf1, idxslab, onesbuf, tbuf, tidx,
                  zrow, accx, acce, accc, sem0, sem1):
    c = lax.axis_index("c")
    s = lax.axis_index("s")
    wid = c * NS + s
    xbufs = (xbuf0, xbuf1)
    ebufs = (ebuf0, ebuf1)
    sems = (sem0, sem1)
    nch = jnp.where(wid == NW - 1, LAST_CHUNKS, SLAB)
    slab0 = wid * SLAB_ROWS

    pltpu.sync_copy(b2d_hbm.at[wid], idxslab)
    pltpu.sync_copy(ones_hbm, onesbuf)

    zvec = jnp.zeros((16,), jnp.float32)
    for i in range(ROWS_PER_TILE):
        for j in range(D // 16):
            zrow[i, pl.ds(j * 16, 16)] = zvec

    base_r = s * ROWS_PER_TILE
    pltpu.sync_copy(zrow, accx.at[pl.ds(base_r, ROWS_PER_TILE)])
    pltpu.sync_copy(zrow, acce.at[pl.ds(base_r, ROWS_PER_TILE)])
    pltpu.sync_copy(zrow, accc.at[pl.ds(base_r, ROWS_PER_TILE)])
    plsc.subcore_barrier()

    def issue(t, b):
        @pl.when(t < nch)
        def _():
            row0 = slab0 + t * CHUNK
            pltpu.async_copy(x_hbm.at[pl.ds(row0, CHUNK)], xbufs[b], sems[b])
            pltpu.async_copy(e_hbm.at[pl.ds(row0, CHUNK)], ebufs[b], sems[b])

    def wait_and_scatter(t, b):
        @pl.when(t < nch)
        def _():
            pltpu.make_async_copy(x_hbm.at[pl.ds(0, CHUNK)], xbufs[b],
                                  sems[b]).wait()
            pltpu.make_async_copy(e_hbm.at[pl.ds(0, CHUNK)], ebufs[b],
                                  sems[b]).wait()
            idx = idxslab.at[t]
            pltpu.sync_copy(xbufs[b], accx.at[idx], add=True)
            pltpu.sync_copy(ebufs[b], acce.at[idx], add=True)
            pltpu.sync_copy(onesbuf, accc.at[idx], add=True)

    issue(0, 0)

    def pair(t2, _):
        for b in range(2):
            t = t2 * 2 + b
            issue(t + 1, 1 - b)
            wait_and_scatter(t, b)
        return 0

    lax.fori_loop(0, SLAB // 2, pair, 0)
    wait_and_scatter(SLAB - 1, (SLAB - 1) % 2)

    @pl.when(wid == NW - 1)
    def _():
        row0 = NUM_FULL * CHUNK
        pltpu.sync_copy(b1d_hbm.at[pl.ds(row0, TAIL)], tidx)
        pltpu.sync_copy(x_hbm.at[pl.ds(row0, TAIL)], tbuf)
        pltpu.sync_copy(tbuf, accx.at[tidx], add=True)
        pltpu.sync_copy(e_hbm.at[pl.ds(row0, TAIL)], tbuf)
        pltpu.sync_copy(tbuf, acce.at[tidx], add=True)
        pltpu.sync_copy(onesbuf.at[pl.ds(0, TAIL)], accc.at[tidx], add=True)

    plsc.subcore_barrier()

    pltpu.sync_copy(accx.at[pl.ds(base_r, ROWS_PER_TILE)], zrow)
    pltpu.sync_copy(zrow, node_out.at[c, pl.ds(base_r, ROWS_PER_TILE)])
    pltpu.sync_copy(acce.at[pl.ds(base_r, ROWS_PER_TILE)], zrow)
    pltpu.sync_copy(zrow, edge_out.at[c, pl.ds(base_r, ROWS_PER_TILE)])
    pltpu.sync_copy(accc.at[pl.ds(base_r, ROWS_PER_TILE)], zrow)
    pltpu.sync_copy(zrow, cnt_out.at[c, pl.ds(base_r, ROWS_PER_TILE)])


def _segment_sums(x, edge_attr, b2d, b1d, ones):
    mesh = plsc.VectorSubcoreMesh(core_axis_name="c", subcore_axis_name="s")
    return pl.kernel(
        _seg_sum_body,
        out_type=(
            jax.ShapeDtypeStruct((NC, B, D), jnp.float32),
            jax.ShapeDtypeStruct((NC, B, D), jnp.float32),
            jax.ShapeDtypeStruct((NC, B, D), jnp.float32),
        ),
        mesh=mesh,
        scratch_types=[
            pltpu.VMEM((CHUNK, D), jnp.float32),
            pltpu.VMEM((CHUNK, D), jnp.float32),
            pltpu.VMEM((CHUNK, D), jnp.float32),
            pltpu.VMEM((CHUNK, D), jnp.float32),
            pltpu.VMEM((SLAB, CHUNK), jnp.int32),
            pltpu.VMEM((CHUNK, D), jnp.float32),
            pltpu.VMEM((TAIL, D), jnp.float32),
            pltpu.VMEM((TAIL,), jnp.int32),
            pltpu.VMEM((ROWS_PER_TILE, D), jnp.float32),
            pltpu.VMEM_SHARED((B, D), jnp.float32),
            pltpu.VMEM_SHARED((B, D), jnp.float32),
            pltpu.VMEM_SHARED((B, D), jnp.float32),
            pltpu.SemaphoreType.DMA,
            pltpu.SemaphoreType.DMA,
        ],
    )(x, edge_attr, b2d, b1d, ones)


def _mlp_body(node_ref, edge_ref, cnt_ref, w1_ref, b1_ref, w2_ref, b2_ref,
              out_ref):
    ns = node_ref[0] + node_ref[1]
    es = edge_ref[0] + edge_ref[1]
    denom = cnt_ref[0] + cnt_ref[1] + 1e-6
    nm = ns / denom
    em = es / denom
    w1 = w1_ref[...]
    dn = (((1,), (1,)), ((), ()))
    h = lax.dot_general(nm, w1[:, :D], dn,
                        preferred_element_type=jnp.float32,
                        precision=lax.Precision.HIGHEST)
    h += lax.dot_general(em, w1[:, D:], dn,
                         preferred_element_type=jnp.float32,
                         precision=lax.Precision.HIGHEST)
    h = jnp.maximum(h + b1_ref[...], 0.0)
    out = lax.dot_general(h, w2_ref[...], dn,
                          preferred_element_type=jnp.float32,
                          precision=lax.Precision.HIGHEST)
    out_ref[...] = out + b2_ref[...]


def _pooled_mlp(node_sums, edge_sums, cnts, W1, b1, W2, b2):
    return pl.pallas_call(
        _mlp_body,
        out_shape=jax.ShapeDtypeStruct((B, D), jnp.float32),
    )(node_sums, edge_sums, cnts, W1, b1.reshape(1, HIDDEN), W2,
      b2.reshape(1, D))


@jax.jit
def kernel(x, edge_index, edge_attr, u, batch, W1, b1, W2, b2):
    del edge_index, u
    batch = batch.astype(jnp.int32)
    b2d = jnp.concatenate(
        [batch, jnp.zeros((NW * SLAB_ROWS - N,), jnp.int32)]
    ).reshape(NW, SLAB, CHUNK)
    ones = jnp.ones((CHUNK, D), jnp.float32)
    node_sums, edge_sums, cnts = _segment_sums(x, edge_attr, b2d, batch, ones)
    return _pooled_mlp(node_sums, edge_sums, cnts, W1, b1, W2, b2)


# speedup vs baseline: 1.0957x; 1.0957x over previous
import jax
import jax.numpy as jnp
from jax import lax
from jax.experimental import pallas as pl
from jax.experimental.pallas import tpu as pltpu
from jax.experimental.pallas import tpu_sc as plsc

N = 100000
D = 128
B = 256
HIDDEN = 256

CHUNK = 128
NUM_FULL = N // CHUNK
TAIL = N - NUM_FULL * CHUNK
NC = 2
NS = 16
NW = NC * NS
MAX_ITERS = (NUM_FULL + NW - 1) // NW
ROWS_PER_TILE = B // NS


def _seg_sum_body(x_hbm, e_hbm, b_hbm, ones_hbm, node_out, edge_out, cnt_out,
                  xbuf0, xbuf1, ebuf0, ebuf1, idx0, idx1, onesbuf, tbuf, tidx,
                  zrow, accx, acce, accc, sem0, sem1):
    c = lax.axis_index("c")
    s = lax.axis_index("s")
    wid = c * NS + s
    xbufs = (xbuf0, xbuf1)
    ebufs = (ebuf0, ebuf1)
    idxs = (idx0, idx1)
    sems = (sem0, sem1)

    pltpu.sync_copy(ones_hbm, onesbuf)

    zvec = jnp.zeros((16,), jnp.float32)
    for i in range(ROWS_PER_TILE):
        for j in range(D // 16):
            zrow[i, pl.ds(j * 16, 16)] = zvec

    base_r = s * ROWS_PER_TILE
    pltpu.sync_copy(zrow, accx.at[pl.ds(base_r, ROWS_PER_TILE)])
    pltpu.sync_copy(zrow, acce.at[pl.ds(base_r, ROWS_PER_TILE)])
    pltpu.sync_copy(zrow, accc.at[pl.ds(base_r, ROWS_PER_TILE)])
    plsc.subcore_barrier()

    def issue(t, b):
        cid = wid + t * NW

        @pl.when(cid < NUM_FULL)
        def _():
            row0 = cid * CHUNK
            pltpu.async_copy(b_hbm.at[pl.ds(row0, CHUNK)], idxs[b], sems[b])
            pltpu.async_copy(x_hbm.at[pl.ds(row0, CHUNK)], xbufs[b], sems[b])
            pltpu.async_copy(e_hbm.at[pl.ds(row0, CHUNK)], ebufs[b], sems[b])

    def wait_and_scatter(t, b):
        cid = wid + t * NW

        @pl.when(cid < NUM_FULL)
        def _():
            pltpu.make_async_copy(b_hbm.at[pl.ds(0, CHUNK)], idxs[b],
                                  sems[b]).wait()
            pltpu.make_async_copy(x_hbm.at[pl.ds(0, CHUNK)], xbufs[b],
                                  sems[b]).wait()
            pltpu.make_async_copy(e_hbm.at[pl.ds(0, CHUNK)], ebufs[b],
                                  sems[b]).wait()
            pltpu.sync_copy(xbufs[b], accx.at[idxs[b]], add=True)
            pltpu.sync_copy(ebufs[b], acce.at[idxs[b]], add=True)
            pltpu.sync_copy(onesbuf, accc.at[idxs[b]], add=True)

    issue(0, 0)

    def pair(t2, _):
        for b in range(2):
            t = t2 * 2 + b
            issue(t + 1, 1 - b)
            wait_and_scatter(t, b)
        return 0

    lax.fori_loop(0, MAX_ITERS // 2, pair, 0)
    wait_and_scatter(MAX_ITERS - 1, (MAX_ITERS - 1) % 2)

    @pl.when(wid == 0)
    def _():
        row0 = NUM_FULL * CHUNK
        pltpu.sync_copy(b_hbm.at[pl.ds(row0, TAIL)], tidx)
        pltpu.sync_copy(x_hbm.at[pl.ds(row0, TAIL)], tbuf)
        pltpu.sync_copy(tbuf, accx.at[tidx], add=True)
        pltpu.sync_copy(e_hbm.at[pl.ds(row0, TAIL)], tbuf)
        pltpu.sync_copy(tbuf, acce.at[tidx], add=True)
        pltpu.sync_copy(onesbuf.at[pl.ds(0, TAIL)], accc.at[tidx], add=True)

    plsc.subcore_barrier()

    pltpu.sync_copy(accx.at[pl.ds(base_r, ROWS_PER_TILE)], zrow)
    pltpu.sync_copy(zrow, node_out.at[c, pl.ds(base_r, ROWS_PER_TILE)])
    pltpu.sync_copy(acce.at[pl.ds(base_r, ROWS_PER_TILE)], zrow)
    pltpu.sync_copy(zrow, edge_out.at[c, pl.ds(base_r, ROWS_PER_TILE)])
    pltpu.sync_copy(accc.at[pl.ds(base_r, ROWS_PER_TILE)], zrow)
    pltpu.sync_copy(zrow, cnt_out.at[c, pl.ds(base_r, ROWS_PER_TILE)])


def _segment_sums(x, edge_attr, batch, ones):
    mesh = plsc.VectorSubcoreMesh(core_axis_name="c", subcore_axis_name="s")
    return pl.kernel(
        _seg_sum_body,
        out_type=(
            jax.ShapeDtypeStruct((NC, B, D), jnp.float32),
            jax.ShapeDtypeStruct((NC, B, D), jnp.float32),
            jax.ShapeDtypeStruct((NC, B, D), jnp.float32),
        ),
        mesh=mesh,
        scratch_types=[
            pltpu.VMEM((CHUNK, D), jnp.float32),
            pltpu.VMEM((CHUNK, D), jnp.float32),
            pltpu.VMEM((CHUNK, D), jnp.float32),
            pltpu.VMEM((CHUNK, D), jnp.float32),
            pltpu.VMEM((CHUNK,), jnp.int32),
            pltpu.VMEM((CHUNK,), jnp.int32),
            pltpu.VMEM((CHUNK, D), jnp.float32),
            pltpu.VMEM((TAIL, D), jnp.float32),
            pltpu.VMEM((TAIL,), jnp.int32),
            pltpu.VMEM((ROWS_PER_TILE, D), jnp.float32),
            pltpu.VMEM_SHARED((B, D), jnp.float32),
            pltpu.VMEM_SHARED((B, D), jnp.float32),
            pltpu.VMEM_SHARED((B, D), jnp.float32),
            pltpu.SemaphoreType.DMA,
            pltpu.SemaphoreType.DMA,
        ],
    )(x, edge_attr, batch, ones)


def _mlp_body(node_ref, edge_ref, cnt_ref, w1_ref, b1_ref, w2_ref, b2_ref,
              out_ref):
    ns = node_ref[0] + node_ref[1]
    es = edge_ref[0] + edge_ref[1]
    denom = cnt_ref[0] + cnt_ref[1] + 1e-6
    nm = ns / denom
    em = es / denom
    w1 = w1_ref[...]
    dn = (((1,), (1,)), ((), ()))
    h = lax.dot_general(nm, w1[:, :D], dn,
                        preferred_element_type=jnp.float32,
                        precision=lax.Precision.HIGHEST)
    h += lax.dot_general(em, w1[:, D:], dn,
                         preferred_element_type=jnp.float32,
                         precision=lax.Precision.HIGHEST)
    h = jnp.maximum(h + b1_ref[...], 0.0)
    out = lax.dot_general(h, w2_ref[...], dn,
                          preferred_element_type=jnp.float32,
                          precision=lax.Precision.HIGHEST)
    out_ref[...] = out + b2_ref[...]


def _pooled_mlp(node_sums, edge_sums, cnts, W1, b1, W2, b2):
    return pl.pallas_call(
        _mlp_body,
        out_shape=jax.ShapeDtypeStruct((B, D), jnp.float32),
    )(node_sums, edge_sums, cnts, W1, b1.reshape(1, HIDDEN), W2,
      b2.reshape(1, D))


@jax.jit
def kernel(x, edge_index, edge_attr, u, batch, W1, b1, W2, b2):
    del edge_index, u
    batch = batch.astype(jnp.int32)
    ones = jnp.ones((CHUNK, D), jnp.float32)
    node_sums, edge_sums, cnts = _segment_sums(x, edge_attr, batch, ones)
    return _pooled_mlp(node_sums, edge_sums, cnts, W1, b1, W2, b2)


# speedup vs baseline: 1.4527x; 1.3258x over previous
import jax
import jax.numpy as jnp
from jax import lax
from jax.experimental import pallas as pl
from jax.experimental.pallas import tpu as pltpu
from jax.experimental.pallas import tpu_sc as plsc

N = 100000
D = 128
B = 256
HIDDEN = 256

CHUNK = 128
NUM_FULL = N // CHUNK
TAIL = N - NUM_FULL * CHUNK
NC = 2
NS = 16
NW = NC * NS
MAX_ITERS = (NUM_FULL + NW - 1) // NW
ROWS_PER_TILE = B // NS
NPAD = NUM_FULL * CHUNK + CHUNK


def _seg_sum_body(x_hbm, e_hbm, b_hbm, node_out, edge_out,
                  xbuf0, xbuf1, xbuf2, ebuf0, ebuf1, ebuf2,
                  idx0, idx1, idx2, tbuf, tidx,
                  zrow, accx, acce, sem0, sem1, sem2):
    c = lax.axis_index("c")
    s = lax.axis_index("s")
    wid = c * NS + s
    xbufs = (xbuf0, xbuf1, xbuf2)
    ebufs = (ebuf0, ebuf1, ebuf2)
    idxs = (idx0, idx1, idx2)
    sems = (sem0, sem1, sem2)

    zvec = jnp.zeros((16,), jnp.float32)
    for i in range(ROWS_PER_TILE):
        for j in range(D // 16):
            zrow[i, pl.ds(j * 16, 16)] = zvec

    base_r = s * ROWS_PER_TILE
    pltpu.sync_copy(zrow, accx.at[pl.ds(base_r, ROWS_PER_TILE)])
    pltpu.sync_copy(zrow, acce.at[pl.ds(base_r, ROWS_PER_TILE)])
    plsc.subcore_barrier()

    def issue(t, b):
        cid = wid + t * NW

        @pl.when(cid < NUM_FULL)
        def _():
            row0 = cid * CHUNK
            pltpu.async_copy(b_hbm.at[pl.ds(row0, CHUNK)], idxs[b], sems[b])
            pltpu.async_copy(x_hbm.at[pl.ds(row0, CHUNK)], xbufs[b], sems[b])
            pltpu.async_copy(e_hbm.at[pl.ds(row0, CHUNK)], ebufs[b], sems[b])

    def wait_and_scatter(t, b):
        cid = wid + t * NW

        @pl.when(cid < NUM_FULL)
        def _():
            pltpu.make_async_copy(b_hbm.at[pl.ds(0, CHUNK)], idxs[b],
                                  sems[b]).wait()
            pltpu.make_async_copy(x_hbm.at[pl.ds(0, CHUNK)], xbufs[b],
                                  sems[b]).wait()
            pltpu.make_async_copy(e_hbm.at[pl.ds(0, CHUNK)], ebufs[b],
                                  sems[b]).wait()
            pltpu.sync_copy(xbufs[b], accx.at[idxs[b]], add=True)
            pltpu.sync_copy(ebufs[b], acce.at[idxs[b]], add=True)

    issue(0, 0)
    issue(1, 1)

    def triple(t3, _):
        for b in range(3):
            t = t3 * 3 + b
            issue(t + 2, (b + 2) % 3)
            wait_and_scatter(t, b)
        return 0

    lax.fori_loop(0, MAX_ITERS // 3, triple, 0)
    wait_and_scatter(MAX_ITERS - 1, (MAX_ITERS - 1) % 3)

    @pl.when(wid == 0)
    def _():
        row0 = NUM_FULL * CHUNK
        pltpu.sync_copy(b_hbm.at[pl.ds(row0, TAIL)], tidx)
        pltpu.sync_copy(x_hbm.at[pl.ds(row0, TAIL)], tbuf)
        pltpu.sync_copy(tbuf, accx.at[tidx], add=True)
        pltpu.sync_copy(e_hbm.at[pl.ds(row0, TAIL)], tbuf)
        pltpu.sync_copy(tbuf, acce.at[tidx], add=True)

    plsc.subcore_barrier()

    pltpu.sync_copy(accx.at[pl.ds(base_r, ROWS_PER_TILE)], zrow)
    pltpu.sync_copy(zrow, node_out.at[c, pl.ds(base_r, ROWS_PER_TILE)])
    pltpu.sync_copy(acce.at[pl.ds(base_r, ROWS_PER_TILE)], zrow)
    pltpu.sync_copy(zrow, edge_out.at[c, pl.ds(base_r, ROWS_PER_TILE)])


def _segment_sums(x, edge_attr, batch):
    mesh = plsc.VectorSubcoreMesh(core_axis_name="c", subcore_axis_name="s")
    return pl.kernel(
        _seg_sum_body,
        out_type=(
            jax.ShapeDtypeStruct((NC, B, D), jnp.float32),
            jax.ShapeDtypeStruct((NC, B, D), jnp.float32),
        ),
        mesh=mesh,
        scratch_types=[
            pltpu.VMEM((CHUNK, D), jnp.float32),
            pltpu.VMEM((CHUNK, D), jnp.float32),
            pltpu.VMEM((CHUNK, D), jnp.float32),
            pltpu.VMEM((CHUNK, D), jnp.float32),
            pltpu.VMEM((CHUNK, D), jnp.float32),
            pltpu.VMEM((CHUNK, D), jnp.float32),
            pltpu.VMEM((CHUNK,), jnp.int32),
            pltpu.VMEM((CHUNK,), jnp.int32),
            pltpu.VMEM((CHUNK,), jnp.int32),
            pltpu.VMEM((TAIL, D), jnp.float32),
            pltpu.VMEM((TAIL,), jnp.int32),
            pltpu.VMEM((ROWS_PER_TILE, D), jnp.float32),
            pltpu.VMEM_SHARED((B, D), jnp.float32),
            pltpu.VMEM_SHARED((B, D), jnp.float32),
            pltpu.SemaphoreType.DMA,
            pltpu.SemaphoreType.DMA,
            pltpu.SemaphoreType.DMA,
        ],
    )(x, edge_attr, batch)


def _hist_body(b_ref, cnt_ref):
    ids = lax.broadcasted_iota(jnp.int32, (B, CHUNK), 0)

    def step(t, acc):
        row = b_ref[pl.ds(t, 1), :]
        return acc + jnp.where(row == ids, 1.0, 0.0)

    cnt_ref[...] = lax.fori_loop(0, NPAD // CHUNK,
                                 step, jnp.zeros((B, CHUNK), jnp.float32))


def _histogram(batch_padded):
    return pl.pallas_call(
        _hist_body,
        out_shape=jax.ShapeDtypeStruct((B, CHUNK), jnp.float32),
    )(batch_padded)


def _mlp_body(node_ref, edge_ref, cnt_ref, w1_ref, b1_ref, w2_ref, b2_ref,
              out_ref):
    ns = node_ref[0] + node_ref[1]
    es = edge_ref[0] + edge_ref[1]
    cnt = jnp.sum(cnt_ref[...], axis=1, keepdims=True)
    denom = cnt + 1e-6
    nm = ns / denom
    em = es / denom
    w1 = w1_ref[...]
    dn = (((1,), (1,)), ((), ()))
    h = lax.dot_general(nm, w1[:, :D], dn,
                        preferred_element_type=jnp.float32,
                        precision=lax.Precision.HIGHEST)
    h += lax.dot_general(em, w1[:, D:], dn,
                         preferred_element_type=jnp.float32,
                         precision=lax.Precision.HIGHEST)
    h = jnp.maximum(h + b1_ref[...], 0.0)
    out = lax.dot_general(h, w2_ref[...], dn,
                          preferred_element_type=jnp.float32,
                          precision=lax.Precision.HIGHEST)
    out_ref[...] = out + b2_ref[...]


def _pooled_mlp(node_sums, edge_sums, cnts, W1, b1, W2, b2):
    return pl.pallas_call(
        _mlp_body,
        out_shape=jax.ShapeDtypeStruct((B, D), jnp.float32),
    )(node_sums, edge_sums, cnts, W1, b1.reshape(1, HIDDEN), W2,
      b2.reshape(1, D))


@jax.jit
def kernel(x, edge_index, edge_attr, u, batch, W1, b1, W2, b2):
    del edge_index, u
    batch = batch.astype(jnp.int32)
    bp = jnp.concatenate([batch, jnp.full((NPAD - N,), B, jnp.int32)])
    cnts = _histogram(bp.reshape(NPAD // CHUNK, CHUNK))
    node_sums, edge_sums = _segment_sums(x, edge_attr, batch)
    return _pooled_mlp(node_sums, edge_sums, cnts, W1, b1, W2, b2)
